# Initial kernel scaffold; baseline (speedup 1.0000x reference)
import jax
import jax.numpy as jnp
from jax.experimental import pallas as pl


def kernel(x, edge_index, edge_attr, z_graph, y_base, batch, enc_w1, enc_b1, enc_w2, enc_b2, g1_ew, g1_eb, g1_w1, g1_b1, g1_w2, g1_b2, g2_ew, g2_eb, g2_w1, g2_b1, g2_w2, g2_b2, syn_w1, syn_b1, syn_w2, syn_b2, conf_w1, conf_b1, conf_w2, conf_b2, bal_w1, bal_b1, bal_w2, bal_b2):
    raise NotImplementedError("write your pallas kernel here")


# speedup vs baseline: 1.0732x; 1.0732x over previous
import jax
import jax.numpy as jnp
from jax.experimental import pallas as pl

ALPHA = 0.3521
BETA = 0.3067
GAMMA = 0.3412


def _assemble_kernel(hg_ref, d_ref, yb_ref, out_ref):
    hg = hg_ref[...]
    d = d_ref[...]
    yb = yb_ref[...]
    d_syn, d_conf, d_bal = d[0, 0], d[0, 1], d[0, 2]
    syn_b, conf_b, bal_b = yb[0, 0], yb[0, 1], yb[0, 2]
    sqe_b = ALPHA * syn_b - BETA * conf_b + GAMMA * bal_b
    d_sqe = ALPHA * d_syn - BETA * d_conf + GAMMA * d_bal
    sqe_c = jax.nn.sigmoid(sqe_b + d_sqe)
    head = jnp.stack([d_syn, d_conf, d_bal, syn_b + d_syn, conf_b + d_conf,
                      bal_b + d_bal, sqe_c])[None, :]
    out_ref[...] = jnp.concatenate([head, hg], axis=1)


def _gine(h, edge_index, edge_attr, ew, eb, w1, b1, w2, b2):
    e = edge_attr @ ew + eb
    m = jax.nn.relu(h[edge_index[0]] + e)
    aggr = jax.ops.segment_sum(m, edge_index[1], num_segments=h.shape[0])
    out = h + aggr
    out = jax.nn.relu(out @ w1 + b1) @ w2 + b2
    return jax.nn.relu(out)


def kernel(x, edge_index, edge_attr, z_graph, y_base, batch, enc_w1, enc_b1, enc_w2, enc_b2, g1_ew, g1_eb, g1_w1, g1_b1, g1_w2, g1_b2, g2_ew, g2_eb, g2_w1, g2_b1, g2_w2, g2_b2, syn_w1, syn_b1, syn_w2, syn_b2, conf_w1, conf_b1, conf_w2, conf_b2, bal_w1, bal_b1, bal_w2, bal_b2):
    h = jax.nn.relu(x @ enc_w1 + enc_b1)
    h = jax.nn.relu(h @ enc_w2 + enc_b2)
    h = _gine(h, edge_index, edge_attr, g1_ew, g1_eb, g1_w1, g1_b1, g1_w2, g1_b2)
    h = _gine(h, edge_index, edge_attr, g2_ew, g2_eb, g2_w1, g2_b1, g2_w2, g2_b2)
    n = h.shape[0]
    h_mean = jnp.sum(h, axis=0, keepdims=True) / n
    h_max = jnp.max(h, axis=0, keepdims=True)
    h_g = jnp.concatenate([h_mean, h_max, z_graph], axis=1)

    def _head(hg, w1, b1, w2, b2):
        return (jax.nn.relu(hg @ w1 + b1) @ w2 + b2)[:, 0]

    d_syn = _head(h_g, syn_w1, syn_b1, syn_w2, syn_b2)
    d_conf = _head(h_g, conf_w1, conf_b1, conf_w2, conf_b2)
    d_bal = _head(h_g, bal_w1, bal_b1, bal_w2, bal_b2)
    d = jnp.stack([d_syn, d_conf, d_bal], axis=1)
    out = pl.pallas_call(
        _assemble_kernel,
        out_shape=jax.ShapeDtypeStruct((1, 7 + h_g.shape[1]), jnp.float32),
    )(h_g, d, y_base)
    return out


# speedup vs baseline: 2.4671x; 2.2988x over previous
import jax
import jax.numpy as jnp
from jax import lax
from jax.experimental import pallas as pl
from jax.experimental.pallas import tpu as pltpu
from jax.experimental.pallas import tpu_sc as plsc

ALPHA = 0.3521
BETA = 0.3067
GAMMA = 0.3412

N = 50000
E = 800000
E_PAD = 802816
H = 64
HH = 32
CHUNK = 512
NCHUNK = 98
EPT = CHUNK * NCHUNK
AGG_ROWS = 50048
ZPT = AGG_ROWS // 16


def _enc_body(x_ref, w1, b1, w2, b2, lo, hi):
    t = jnp.maximum(x_ref[...] @ w1[...] + b1[...], 0.0)
    h = jnp.maximum(t @ w2[...] + b2[...], 0.0)
    lo[...] = h[:, :HH]
    hi[...] = h[:, HH:]


def _mlp_body(lo, hi, alo, ahi, w1, b1, w2, b2, olo, ohi):
    h = jnp.concatenate([lo[...], hi[...]], axis=1) + \
        jnp.concatenate([alo[...], ahi[...]], axis=1)
    t = jnp.maximum(h @ w1[...] + b1[...], 0.0)
    o = jnp.maximum(t @ w2[...] + b2[...], 0.0)
    olo[...] = o[:, :HH]
    ohi[...] = o[:, HH:]


def _mlp_pool_body(lo, hi, alo, ahi, w1, b1, w2, b2, s_ref, m_ref):
    h = jnp.concatenate([lo[...], hi[...]], axis=1) + \
        jnp.concatenate([alo[...], ahi[...]], axis=1)
    t = jnp.maximum(h @ w1[...] + b1[...], 0.0)
    o = jnp.maximum(t @ w2[...] + b2[...], 0.0)
    o3 = o.reshape(50, 8, H)
    s = jnp.sum(o3, axis=0)
    m = jnp.max(o3, axis=0)

    @pl.when(pl.program_id(0) == 0)
    def _():
        s_ref[...] = s
        m_ref[...] = m

    @pl.when(pl.program_id(0) > 0)
    def _():
        s_ref[...] = s_ref[...] + s
        m_ref[...] = jnp.maximum(m_ref[...], m)


def _heads_body(s_ref, m_ref, z_ref, yb_ref,
                sw1, sb1, sw2, sb2, cw1, cb1, cw2, cb2, bw1, bb1, bw2, bb2,
                out_ref):
    mean = jnp.sum(s_ref[...], axis=0, keepdims=True) * (1.0 / N)
    mx = jnp.max(m_ref[...], axis=0, keepdims=True)
    hg = jnp.concatenate([mean, mx, z_ref[...]], axis=1)

    def head(w1, b1, w2, b2):
        t = jnp.maximum(hg @ w1[...] + b1[...], 0.0)
        return (t @ w2[...] + b2[...])[0, 0]

    d_syn = head(sw1, sb1, sw2, sb2)
    d_conf = head(cw1, cb1, cw2, cb2)
    d_bal = head(bw1, bb1, bw2, bb2)
    yb = yb_ref[...]
    syn_b, conf_b, bal_b = yb[0, 0], yb[0, 1], yb[0, 2]
    sqe_b = ALPHA * syn_b - BETA * conf_b + GAMMA * bal_b
    d_sqe = ALPHA * d_syn - BETA * d_conf + GAMMA * d_bal
    sqe_c = jax.nn.sigmoid(sqe_b + d_sqe)
    hd = jnp.stack([d_syn, d_conf, d_bal, syn_b + d_syn, conf_b + d_conf,
                    bal_b + d_bal, sqe_c])[None, :]
    out_ref[...] = jnp.concatenate([hd, hg], axis=1)


def _sc_body(h_lo, h_hi, attr2, src2, dst2, ew_in, eb_in, alo, ahi,
             src_v, dst_v, attr_v, rows_v, ew_v, eb_v, aggr_sp, sem):
    c = lax.axis_index("c")
    s = lax.axis_index("s")

    pltpu.sync_copy(ew_in, ew_v)
    pltpu.sync_copy(eb_in, eb_v)

    def zb_body(i, _):
        rows_v[i, pl.ds(0, 16)] = jnp.zeros((16,), jnp.float32)
        rows_v[i, pl.ds(16, 16)] = jnp.zeros((16,), jnp.float32)
        return 0
    lax.fori_loop(0, CHUNK, zb_body, 0)
    zb = s * ZPT
    for off in range(0, 3072, CHUNK):
        pltpu.sync_copy(rows_v.at[pl.ds(0, CHUNK)], aggr_sp.at[pl.ds(zb + off, CHUNK)])
    pltpu.sync_copy(rows_v.at[pl.ds(0, ZPT - 3072)], aggr_sp.at[pl.ds(zb + 3072, ZPT - 3072)])
    plsc.subcore_barrier()

    def run(h_c, out_c, cb_base):
        w = [(ew_v[k, pl.ds(cb_base, 16)], ew_v[k, pl.ds(cb_base + 16, 16)])
             for k in range(4)]
        b0 = eb_v[0, pl.ds(cb_base, 16)]
        b1 = eb_v[0, pl.ds(cb_base + 16, 16)]

        def chunk_body(ch, carry):
            ib = s * (EPT // 128) + ch * 4
            ab = s * (EPT // 32) + ch * 16
            pltpu.sync_copy(src2.at[pl.ds(ib, 4)], src_v)
            pltpu.sync_copy(dst2.at[pl.ds(ib, 4)], dst_v)
            pltpu.sync_copy(attr2.at[pl.ds(ab, 16)], attr_v)
            cps = []
            for j in range(4):
                cp = pltpu.make_async_copy(
                    h_c.at[src_v.at[j]], rows_v.at[pl.ds(j * 128, 128)], sem)
                cp.start()
                cps.append(cp)
            for cp in cps:
                cp.wait()

            def cb(t, cy):
                (w00, w01), (w10, w11), (w20, w21), (w30, w31), c0, c1 = cy
                av = attr_v[t // 8, pl.ds((t % 8) * 16, 16)]
                for sub in range(4):
                    i = t * 4 + sub
                    a0 = av[sub * 4 + 0]
                    a1 = av[sub * 4 + 1]
                    a2 = av[sub * 4 + 2]
                    a3 = av[sub * 4 + 3]
                    acc0 = c0 + a0 * w00 + a1 * w10 + a2 * w20 + a3 * w30
                    acc1 = c1 + a0 * w01 + a1 * w11 + a2 * w21 + a3 * w31
                    g0 = rows_v[i, pl.ds(0, 16)]
                    g1 = rows_v[i, pl.ds(16, 16)]
                    rows_v[i, pl.ds(0, 16)] = jnp.maximum(g0 + acc0, 0.0)
                    rows_v[i, pl.ds(16, 16)] = jnp.maximum(g1 + acc1, 0.0)
                return cy
            lax.fori_loop(0, CHUNK // 4, cb, carry)

            for j in range(4):
                pltpu.sync_copy(rows_v.at[pl.ds(j * 128, 128)],
                                aggr_sp.at[dst_v.at[j]], add=True)
            return carry
        lax.fori_loop(0, NCHUNK, chunk_body,
                      (w[0], w[1], w[2], w[3], b0, b1))
        plsc.subcore_barrier()
        wb = s * ZPT

        @pl.when(s < 15)
        def _():
            pltpu.sync_copy(aggr_sp.at[pl.ds(wb, ZPT)], out_c.at[pl.ds(wb, ZPT)])

        @pl.when(s == 15)
        def _():
            pltpu.sync_copy(aggr_sp.at[pl.ds(15 * ZPT, N - 15 * ZPT)],
                            out_c.at[pl.ds(15 * ZPT, N - 15 * ZPT)])

    @pl.when(c == 0)
    def _():
        run(h_lo, alo, 0)

    @pl.when(c == 1)
    def _():
        run(h_hi, ahi, HH)


def _sc_gine(h_lo, h_hi, attr2, src2, dst2, ew, eb):
    f32 = jnp.float32
    return pl.kernel(
        _sc_body,
        out_type=[jax.ShapeDtypeStruct((N, HH), f32),
                  jax.ShapeDtypeStruct((N, HH), f32)],
        mesh=plsc.VectorSubcoreMesh(core_axis_name="c", subcore_axis_name="s"),
        scratch_types=[
            pltpu.VMEM((4, 128), jnp.int32),
            pltpu.VMEM((4, 128), jnp.int32),
            pltpu.VMEM((16, 128), f32),
            pltpu.VMEM((CHUNK, HH), f32),
            pltpu.VMEM((4, H), f32),
            pltpu.VMEM((1, H), f32),
            pltpu.VMEM_SHARED((AGG_ROWS, HH), f32),
            pltpu.SemaphoreType.DMA,
        ],
        compiler_params=pltpu.CompilerParams(use_tc_tiling_on_sc=False),
    )(h_lo, h_hi, attr2, src2, dst2, ew, eb)


def kernel(x, edge_index, edge_attr, z_graph, y_base, batch, enc_w1, enc_b1, enc_w2, enc_b2, g1_ew, g1_eb, g1_w1, g1_b1, g1_w2, g1_b2, g2_ew, g2_eb, g2_w1, g2_b1, g2_w2, g2_b2, syn_w1, syn_b1, syn_w2, syn_b2, conf_w1, conf_b1, conf_w2, conf_b2, bal_w1, bal_b1, bal_w2, bal_b2):
    f32 = jnp.float32
    sds = jax.ShapeDtypeStruct

    pad = E_PAD - E
    src2 = jnp.pad(edge_index[0], (0, pad)).reshape(E_PAD // 128, 128)
    dst2 = jnp.pad(edge_index[1], (0, pad), constant_values=N).reshape(E_PAD // 128, 128)
    attr2 = jnp.pad(edge_attr, ((0, pad), (0, 0))).reshape(E_PAD // 32, 128)

    h_lo, h_hi = pl.pallas_call(
        _enc_body,
        grid=(50,),
        in_specs=[
            pl.BlockSpec((1000, 128), lambda i: (i, 0)),
            pl.BlockSpec((128, H), lambda i: (0, 0)),
            pl.BlockSpec((1, H), lambda i: (0, 0)),
            pl.BlockSpec((H, H), lambda i: (0, 0)),
            pl.BlockSpec((1, H), lambda i: (0, 0)),
        ],
        out_specs=[pl.BlockSpec((1000, HH), lambda i: (i, 0))] * 2,
        out_shape=[sds((N, HH), f32)] * 2,
    )(x, enc_w1, enc_b1[None, :], enc_w2, enc_b2[None, :])

    a1_lo, a1_hi = _sc_gine(h_lo, h_hi, attr2, src2, dst2, g1_ew, g1_eb[None, :])
    mlp_specs = [
        pl.BlockSpec((400, HH), lambda i: (i, 0)),
        pl.BlockSpec((400, HH), lambda i: (i, 0)),
        pl.BlockSpec((400, HH), lambda i: (i, 0)),
        pl.BlockSpec((400, HH), lambda i: (i, 0)),
        pl.BlockSpec((H, H), lambda i: (0, 0)),
        pl.BlockSpec((1, H), lambda i: (0, 0)),
        pl.BlockSpec((H, H), lambda i: (0, 0)),
        pl.BlockSpec((1, H), lambda i: (0, 0)),
    ]
    h1_lo, h1_hi = pl.pallas_call(
        _mlp_body,
        grid=(125,),
        in_specs=mlp_specs,
        out_specs=[pl.BlockSpec((400, HH), lambda i: (i, 0))] * 2,
        out_shape=[sds((N, HH), f32)] * 2,
    )(h_lo, h_hi, a1_lo, a1_hi, g1_w1, g1_b1[None, :], g1_w2, g1_b2[None, :])

    a2_lo, a2_hi = _sc_gine(h1_lo, h1_hi, attr2, src2, dst2, g2_ew, g2_eb[None, :])
    sum8, max8 = pl.pallas_call(
        _mlp_pool_body,
        grid=(125,),
        in_specs=mlp_specs,
        out_specs=[pl.BlockSpec((8, H), lambda i: (0, 0))] * 2,
        out_shape=[sds((8, H), f32)] * 2,
    )(h1_lo, h1_hi, a2_lo, a2_hi, g2_w1, g2_b1[None, :], g2_w2, g2_b2[None, :])

    out = pl.pallas_call(
        _heads_body,
        out_shape=sds((1, 151), f32),
    )(sum8, max8, z_graph, y_base,
      syn_w1, syn_b1[None, :], syn_w2, syn_b2[None, :],
      conf_w1, conf_b1[None, :], conf_w2, conf_b2[None, :],
      bal_w1, bal_b1[None, :], bal_w2, bal_b2[None, :])
    return out


# speedup vs baseline: 4.4104x; 1.7877x over previous
import jax
import jax.numpy as jnp
from jax import lax
from jax.experimental import pallas as pl
from jax.experimental.pallas import tpu as pltpu
from jax.experimental.pallas import tpu_sc as plsc

ALPHA = 0.3521
BETA = 0.3067
GAMMA = 0.3412

N = 50000
E = 800000
H = 64
HH = 32
NBLK = E // 128
BPT = 392
AGG_ROWS = 50048
ZPT = AGG_ROWS // 16


def _enc_body(x_ref, w1, b1, w2, b2, lo, hi):
    t = jnp.maximum(x_ref[...] @ w1[...] + b1[...], 0.0)
    h = jnp.maximum(t @ w2[...] + b2[...], 0.0)
    lo[...] = h[:, :HH]
    hi[...] = h[:, HH:]


def _mlp_body(lo, hi, alo, ahi, w1, b1, w2, b2, olo, ohi):
    h = jnp.concatenate([lo[...], hi[...]], axis=1) + \
        jnp.concatenate([alo[...], ahi[...]], axis=1)
    t = jnp.maximum(h @ w1[...] + b1[...], 0.0)
    o = jnp.maximum(t @ w2[...] + b2[...], 0.0)
    olo[...] = o[:, :HH]
    ohi[...] = o[:, HH:]


def _mlp_pool_body(lo, hi, alo, ahi, w1, b1, w2, b2, s_ref, m_ref):
    h = jnp.concatenate([lo[...], hi[...]], axis=1) + \
        jnp.concatenate([alo[...], ahi[...]], axis=1)
    t = jnp.maximum(h @ w1[...] + b1[...], 0.0)
    o = jnp.maximum(t @ w2[...] + b2[...], 0.0)
    o3 = o.reshape(50, 8, H)
    s = jnp.sum(o3, axis=0)
    m = jnp.max(o3, axis=0)

    @pl.when(pl.program_id(0) == 0)
    def _():
        s_ref[...] = s
        m_ref[...] = m

    @pl.when(pl.program_id(0) > 0)
    def _():
        s_ref[...] = s_ref[...] + s
        m_ref[...] = jnp.maximum(m_ref[...], m)


def _heads_body(s_ref, m_ref, z_ref, yb_ref,
                sw1, sb1, sw2, sb2, cw1, cb1, cw2, cb2, bw1, bb1, bw2, bb2,
                out_ref):
    mean = jnp.sum(s_ref[...], axis=0, keepdims=True) * (1.0 / N)
    mx = jnp.max(m_ref[...], axis=0, keepdims=True)
    hg = jnp.concatenate([mean, mx, z_ref[...]], axis=1)

    def head(w1, b1, w2, b2):
        t = jnp.maximum(hg @ w1[...] + b1[...], 0.0)
        return (t @ w2[...] + b2[...])[0, 0]

    d_syn = head(sw1, sb1, sw2, sb2)
    d_conf = head(cw1, cb1, cw2, cb2)
    d_bal = head(bw1, bb1, bw2, bb2)
    yb = yb_ref[...]
    syn_b, conf_b, bal_b = yb[0, 0], yb[0, 1], yb[0, 2]
    sqe_b = ALPHA * syn_b - BETA * conf_b + GAMMA * bal_b
    d_sqe = ALPHA * d_syn - BETA * d_conf + GAMMA * d_bal
    sqe_c = jax.nn.sigmoid(sqe_b + d_sqe)
    hd = jnp.stack([d_syn, d_conf, d_bal, syn_b + d_syn, conf_b + d_conf,
                    bal_b + d_bal, sqe_c])[None, :]
    out_ref[...] = jnp.concatenate([hd, hg], axis=1)


def _sc_body(h_lo, h_hi, attr_r, ei2, ew_in, eb_in, alo, ahi,
             src_v, dst_v, attr_v, rows_v, ew_v, eb_v, aggr_sp, sem):
    c = lax.axis_index("c")
    s = lax.axis_index("s")

    pltpu.sync_copy(ew_in, ew_v)
    pltpu.sync_copy(eb_in, eb_v)

    def zb_body(i, _):
        rows_v[i, pl.ds(0, 16)] = jnp.zeros((16,), jnp.float32)
        rows_v[i, pl.ds(16, 16)] = jnp.zeros((16,), jnp.float32)
        return 0
    lax.fori_loop(0, 512, zb_body, 0)
    zb = s * ZPT
    for off in range(0, 3072, 512):
        pltpu.sync_copy(rows_v.at[pl.ds(0, 512)], aggr_sp.at[pl.ds(zb + off, 512)])
    pltpu.sync_copy(rows_v.at[pl.ds(0, ZPT - 3072)], aggr_sp.at[pl.ds(zb + 3072, ZPT - 3072)])
    plsc.subcore_barrier()

    def run(h_c, out_c, cb_base):
        w = [(ew_v[k, pl.ds(cb_base, 16)], ew_v[k, pl.ds(cb_base + 16, 16)])
             for k in range(4)]
        b0 = eb_v[0, pl.ds(cb_base, 16)]
        b1 = eb_v[0, pl.ds(cb_base + 16, 16)]
        carry0 = (w[0], w[1], w[2], w[3], b0, b1)

        def chunk_ops(b0r, nb, cy):
            sv = src_v if nb == 4 else src_v.at[pl.ds(0, nb)]
            dv = dst_v if nb == 4 else dst_v.at[pl.ds(0, nb)]
            av = attr_v if nb == 4 else attr_v.at[pl.ds(0, 4 * nb)]
            pltpu.sync_copy(ei2.at[pl.ds(b0r, nb)], sv)
            pltpu.sync_copy(ei2.at[pl.ds(NBLK + b0r, nb)], dv)
            pltpu.sync_copy(attr_r.at[pl.ds(4 * b0r, 4 * nb)], av)
            cps = []
            for j in range(nb):
                cp = pltpu.make_async_copy(
                    h_c.at[src_v.at[j]], rows_v.at[pl.ds(j * 128, 128)], sem)
                cp.start()
                cps.append(cp)
            for cp in cps:
                cp.wait()

            def gb(t, cy2):
                (w00, w01), (w10, w11), (w20, w21), (w30, w31), c0, c1 = cy2
                lo16 = pl.ds((t % 8) * 16, 16)
                v0 = attr_v[(t // 8) * 4 + 0, lo16]
                v1 = attr_v[(t // 8) * 4 + 1, lo16]
                v2 = attr_v[(t // 8) * 4 + 2, lo16]
                v3 = attr_v[(t // 8) * 4 + 3, lo16]
                for sub in range(16):
                    i = t * 16 + sub
                    acc0 = c0 + v0[sub] * w00 + v1[sub] * w10 \
                        + v2[sub] * w20 + v3[sub] * w30
                    acc1 = c1 + v0[sub] * w01 + v1[sub] * w11 \
                        + v2[sub] * w21 + v3[sub] * w31
                    g0 = rows_v[i, pl.ds(0, 16)]
                    g1 = rows_v[i, pl.ds(16, 16)]
                    rows_v[i, pl.ds(0, 16)] = jnp.maximum(g0 + acc0, 0.0)
                    rows_v[i, pl.ds(16, 16)] = jnp.maximum(g1 + acc1, 0.0)
                return cy2
            lax.fori_loop(0, nb * 8, gb, cy)

            for j in range(nb):
                pltpu.sync_copy(rows_v.at[pl.ds(j * 128, 128)],
                                aggr_sp.at[dst_v.at[j]], add=True)

        tb = s * BPT

        def body4(ch, cy):
            chunk_ops(tb + 4 * ch, 4, cy)
            return cy
        cyr = lax.fori_loop(0, 92, body4, carry0)

        @pl.when(s < 15)
        def _():
            lax.fori_loop(92, 98, body4, cyr)

        @pl.when(s == 15)
        def _():
            chunk_ops(15 * BPT + 368, 2, cyr)

        plsc.subcore_barrier()
        wb = s * ZPT

        @pl.when(s < 15)
        def _():
            pltpu.sync_copy(aggr_sp.at[pl.ds(wb, ZPT)], out_c.at[pl.ds(wb, ZPT)])

        @pl.when(s == 15)
        def _():
            pltpu.sync_copy(aggr_sp.at[pl.ds(15 * ZPT, N - 15 * ZPT)],
                            out_c.at[pl.ds(15 * ZPT, N - 15 * ZPT)])

    @pl.when(c == 0)
    def _():
        run(h_lo, alo, 0)

    @pl.when(c == 1)
    def _():
        run(h_hi, ahi, HH)


def _sc_gine(h_lo, h_hi, attr_r, ei2, ew, eb):
    f32 = jnp.float32
    return pl.kernel(
        _sc_body,
        out_type=[jax.ShapeDtypeStruct((N, HH), f32),
                  jax.ShapeDtypeStruct((N, HH), f32)],
        mesh=plsc.VectorSubcoreMesh(core_axis_name="c", subcore_axis_name="s"),
        scratch_types=[
            pltpu.VMEM((4, 128), jnp.int32),
            pltpu.VMEM((4, 128), jnp.int32),
            pltpu.VMEM((16, 128), f32),
            pltpu.VMEM((512, HH), f32),
            pltpu.VMEM((4, H), f32),
            pltpu.VMEM((1, H), f32),
            pltpu.VMEM_SHARED((AGG_ROWS, HH), f32),
            pltpu.SemaphoreType.DMA,
        ],
        compiler_params=pltpu.CompilerParams(use_tc_tiling_on_sc=False),
    )(h_lo, h_hi, attr_r, ei2, ew, eb)


def kernel(x, edge_index, edge_attr, z_graph, y_base, batch, enc_w1, enc_b1, enc_w2, enc_b2, g1_ew, g1_eb, g1_w1, g1_b1, g1_w2, g1_b2, g2_ew, g2_eb, g2_w1, g2_b1, g2_w2, g2_b2, syn_w1, syn_b1, syn_w2, syn_b2, conf_w1, conf_b1, conf_w2, conf_b2, bal_w1, bal_b1, bal_w2, bal_b2):
    f32 = jnp.float32
    sds = jax.ShapeDtypeStruct

    attr_r = edge_attr.reshape(NBLK, 128, 4).transpose(0, 2, 1).reshape(4 * NBLK, 128)
    ei2 = edge_index.reshape(2 * NBLK, 128)

    h_lo, h_hi = pl.pallas_call(
        _enc_body,
        grid=(50,),
        in_specs=[
            pl.BlockSpec((1000, 128), lambda i: (i, 0)),
            pl.BlockSpec((128, H), lambda i: (0, 0)),
            pl.BlockSpec((1, H), lambda i: (0, 0)),
            pl.BlockSpec((H, H), lambda i: (0, 0)),
            pl.BlockSpec((1, H), lambda i: (0, 0)),
        ],
        out_specs=[pl.BlockSpec((1000, HH), lambda i: (i, 0))] * 2,
        out_shape=[sds((N, HH), f32)] * 2,
    )(x, enc_w1, enc_b1[None, :], enc_w2, enc_b2[None, :])

    a1_lo, a1_hi = _sc_gine(h_lo, h_hi, attr_r, ei2, g1_ew, g1_eb[None, :])
    mlp_specs = [
        pl.BlockSpec((400, HH), lambda i: (i, 0)),
        pl.BlockSpec((400, HH), lambda i: (i, 0)),
        pl.BlockSpec((400, HH), lambda i: (i, 0)),
        pl.BlockSpec((400, HH), lambda i: (i, 0)),
        pl.BlockSpec((H, H), lambda i: (0, 0)),
        pl.BlockSpec((1, H), lambda i: (0, 0)),
        pl.BlockSpec((H, H), lambda i: (0, 0)),
        pl.BlockSpec((1, H), lambda i: (0, 0)),
    ]
    h1_lo, h1_hi = pl.pallas_call(
        _mlp_body,
        grid=(125,),
        in_specs=mlp_specs,
        out_specs=[pl.BlockSpec((400, HH), lambda i: (i, 0))] * 2,
        out_shape=[sds((N, HH), f32)] * 2,
    )(h_lo, h_hi, a1_lo, a1_hi, g1_w1, g1_b1[None, :], g1_w2, g1_b2[None, :])

    a2_lo, a2_hi = _sc_gine(h1_lo, h1_hi, attr_r, ei2, g2_ew, g2_eb[None, :])
    sum8, max8 = pl.pallas_call(
        _mlp_pool_body,
        grid=(125,),
        in_specs=mlp_specs,
        out_specs=[pl.BlockSpec((8, H), lambda i: (0, 0))] * 2,
        out_shape=[sds((8, H), f32)] * 2,
    )(h1_lo, h1_hi, a2_lo, a2_hi, g2_w1, g2_b1[None, :], g2_w2, g2_b2[None, :])

    out = pl.pallas_call(
        _heads_body,
        out_shape=sds((1, 151), f32),
    )(sum8, max8, z_graph, y_base,
      syn_w1, syn_b1[None, :], syn_w2, syn_b2[None, :],
      conf_w1, conf_b1[None, :], conf_w2, conf_b2[None, :],
      bal_w1, bal_b1[None, :], bal_w2, bal_b2[None, :])
    return out
